# Initial kernel scaffold; baseline (speedup 1.0000x reference)
import jax
import jax.numpy as jnp
from jax.experimental import pallas as pl


def kernel(x_pf, batch_pf, W1, b1, W2, b2, Wc, bc, Wo1, bo1, Wo2, bo2, Wo3, bo3):
    raise NotImplementedError("write your pallas kernel here")


# speedup vs baseline: 12.2975x; 12.2975x over previous
import functools

import jax
import jax.numpy as jnp
from jax import lax
from jax.experimental import pallas as pl
from jax.experimental.pallas import tpu as pltpu
from jax.experimental.pallas import tpu_sc as plsc

N = 8192
B = 16
K = 8
H = 16
RT = 512
CT = 512
NI = N // RT
NJ = N // CT
IDX_BIG = 2 ** 30


def _elu(x):
    return jnp.where(x > 0, x, jnp.exp(jnp.where(x > 0, 0.0, x)) - 1.0)


def _prep_body(x_ref, w1_ref, b1_ref, w2_ref, b2_ref, wcb_ref, wd_ref,
               bc_ref, h_ref, m_ref, a_ref):
    x = x_ref[...]
    h = _elu(jax.lax.dot_general(x, w1_ref[...], (((1,), (0,)), ((), ())),
                                 preferred_element_type=jnp.float32)
             + b1_ref[...])
    h = _elu(jax.lax.dot_general(h, w2_ref[...], (((1,), (0,)), ((), ())),
                                 preferred_element_type=jnp.float32)
             + b2_ref[...])
    h_ref[...] = h
    m_ref[...] = jax.lax.dot_general(h, wcb_ref[...], (((1,), (0,)), ((), ())),
                                     preferred_element_type=jnp.float32)
    a_ref[...] = jax.lax.dot_general(h, wd_ref[...], (((1,), (0,)), ((), ())),
                                     preferred_element_type=jnp.float32) + bc_ref[...]


def _derive_body(x_ref, wcb_ref, wd_ref, bc_ref, m_ref, a_ref):
    h = x_ref[...]
    m_ref[...] = jax.lax.dot_general(h, wcb_ref[...], (((1,), (0,)), ((), ())),
                                     preferred_element_type=jnp.float32)
    a_ref[...] = jax.lax.dot_general(h, wd_ref[...], (((1,), (0,)), ((), ())),
                                     preferred_element_type=jnp.float32) + bc_ref[...]


def _topk_body(valid_ref, fetch_ref, hrow_ref, hcol_ref, brow_ref, bcolT_ref,
               out_ref, bval, bidx):
    i = pl.program_id(0)
    j = pl.program_id(1)

    @pl.when(j == 0)
    def _init():
        bval[...] = jnp.full((RT, K), jnp.inf, jnp.float32)
        bidx[...] = jnp.full((RT, K), IDX_BIG, jnp.int32)

    step = i * NJ + j
    valid = valid_ref[step]

    @pl.when(valid != 0)
    def _compute():
        hr = hrow_ref[...]
        hc = hcol_ref[...]
        sqr = jnp.sum(hr * hr, axis=1)
        sqc = jnp.sum(hc * hc, axis=1)
        dots = jax.lax.dot_general(hr, hc, (((1,), (1,)), ((), ())),
                                   preferred_element_type=jnp.float32)
        d2 = sqr[:, None] + sqc[None, :] - 2.0 * dots
        cross = brow_ref[...] != bcolT_ref[...]
        d2 = jnp.where(cross, jnp.inf, d2)
        coff = fetch_ref[step] * CT
        citer = jax.lax.broadcasted_iota(jnp.int32, (RT, CT), 1)
        ks = jax.lax.broadcasted_iota(jnp.int32, (RT, K), 1)
        bv = bval[...]
        bi = bidx[...]
        for _ in range(K):
            mv = jnp.min(d2, axis=1)
            im = jnp.where(d2 == mv[:, None], citer, IDX_BIG)
            mi = jnp.min(im, axis=1)
            d2 = jnp.where(citer == mi[:, None], jnp.inf, d2)
            ci = mi + coff
            lt = (bv < mv[:, None]) | ((bv == mv[:, None]) & (bi < ci[:, None]))
            pos = jnp.sum(lt.astype(jnp.int32), axis=1)
            sv = jnp.concatenate([bv[:, :1], bv[:, :K - 1]], axis=1)
            si = jnp.concatenate([bi[:, :1], bi[:, :K - 1]], axis=1)
            keep = ks < pos[:, None]
            new = ks == pos[:, None]
            bv = jnp.where(keep, bv, jnp.where(new, mv[:, None], sv))
            bi = jnp.where(keep, bi, jnp.where(new, ci[:, None], si))
        bval[...] = bv
        bidx[...] = bi

    out_ref[...] = jnp.clip(bidx[...], 0, N - 1)


def _make_topk(interpret=False):
    grid_spec = pltpu.PrefetchScalarGridSpec(
        num_scalar_prefetch=2,
        grid=(NI, NJ),
        in_specs=[
            pl.BlockSpec((RT, H), lambda i, j, v, f: (i, 0)),
            pl.BlockSpec((CT, H), lambda i, j, v, f: (f[i * NJ + j], 0)),
            pl.BlockSpec((RT, 1), lambda i, j, v, f: (i, 0)),
            pl.BlockSpec((1, CT), lambda i, j, v, f: (0, f[i * NJ + j])),
        ],
        out_specs=pl.BlockSpec((RT, K), lambda i, j, v, f: (i, 0)),
        scratch_shapes=[
            pltpu.VMEM((RT, K), jnp.float32),
            pltpu.VMEM((RT, K), jnp.int32),
        ],
    )
    return pl.pallas_call(
        _topk_body,
        grid_spec=grid_spec,
        out_shape=jax.ShapeDtypeStruct((N, K), jnp.int32),
        interpret=interpret,
    )


def _topk_maps(batch):
    bs = batch[::RT]
    be = batch[RT - 1::RT]
    valid = (bs[None, :] <= be[:, None]) & (be[None, :] >= bs[:, None])
    jlo = jnp.argmax(valid, axis=1).astype(jnp.int32)
    jhi = (NJ - 1) - jnp.argmax(valid[:, ::-1], axis=1).astype(jnp.int32)
    fetch = jnp.clip(jnp.arange(NJ, dtype=jnp.int32)[None, :],
                     jlo[:, None], jhi[:, None])
    return valid.astype(jnp.int32).reshape(-1), fetch.reshape(-1)


_NW = 32
_NPW = N // _NW
_CH = 128
_NCHUNK = _NPW * K // _CH
_NPC = _CH // K
_MW = 128


def _sc_gather_body(m_hbm, a_hbm, idx_hbm, out_hbm, idx_v, rows0, rows1, a_v,
                    f_v, sem):
    wid = lax.axis_index("s") * 2 + lax.axis_index("c")
    base = wid * _NPW
    pltpu.sync_copy(idx_hbm.at[pl.ds(base * K, _NPW * K)], idx_v)
    pltpu.sync_copy(a_hbm.at[pl.ds(base * H, _NPW * H)], a_v)
    bufs = (rows0, rows1)

    def fire(c):
        return pltpu.async_copy(
            m_hbm.at[idx_v.at[pl.ds(c * _CH, _CH)]], bufs[c % 2], sem)

    pending = fire(0)
    for c in range(_NCHUNK):
        pending.wait()
        if c + 1 < _NCHUNK:
            pending = fire(c + 1)
        buf = bufs[c % 2]
        for nl in range(_NPC):
            n = c * _NPC + nl
            av = a_v[pl.ds(n * H, H)]
            msg = av + buf[nl * K, pl.ds(0, H)]
            acc = jnp.where(msg > 0, msg, jnp.exp(msg) - 1.0)
            for kk in range(1, K):
                msg = av + buf[nl * K + kk, pl.ds(0, H)]
                acc = jnp.maximum(acc,
                                  jnp.where(msg > 0, msg, jnp.exp(msg) - 1.0))
            f_v[pl.ds(n * H, H)] = acc
    pltpu.sync_copy(f_v, out_hbm.at[pl.ds(base * H, _NPW * H)])


def _sc_gather(m_pad, a_flat, idx_flat):
    mesh = plsc.VectorSubcoreMesh(core_axis_name="c", subcore_axis_name="s")
    fn = functools.partial(
        pl.kernel,
        out_type=jax.ShapeDtypeStruct((N * H,), jnp.float32),
        mesh=mesh,
        scratch_types=[
            pltpu.VMEM((_NPW * K,), jnp.int32),
            pltpu.VMEM((_CH, _MW), jnp.float32),
            pltpu.VMEM((_CH, _MW), jnp.float32),
            pltpu.VMEM((_NPW * H,), jnp.float32),
            pltpu.VMEM((_NPW * H,), jnp.float32),
            pltpu.SemaphoreType.DMA,
        ],
    )(_sc_gather_body)
    return fn(m_pad, a_flat, idx_flat).reshape(N, H)


def _final_body(f2_ref, bT_ref, wo1_ref, bo1_ref, wo2_ref, bo2_ref, wo3_ref,
                bo3_ref, out_ref):
    f2 = f2_ref[...]
    bT = bT_ref[...]
    rows = jax.lax.broadcasted_iota(jnp.int32, (B, N), 0)
    oh = (rows == bT).astype(jnp.float32)
    cnt = jnp.sum(oh, axis=1)
    s = jax.lax.dot_general(oh, f2, (((1,), (0,)), ((), ())),
                            preferred_element_type=jnp.float32)
    pooled = s / jnp.maximum(cnt, 1.0)[:, None]
    o = _elu(jax.lax.dot_general(pooled, wo1_ref[...], (((1,), (0,)), ((), ())),
                                 preferred_element_type=jnp.float32)
             + bo1_ref[...])
    o = _elu(jax.lax.dot_general(o, wo2_ref[...], (((1,), (0,)), ((), ())),
                                 preferred_element_type=jnp.float32)
             + bo2_ref[...])
    o = jax.lax.dot_general(o, wo3_ref[...], (((1,), (0,)), ((), ())),
                            preferred_element_type=jnp.float32) + bo3_ref[...]
    out_ref[...] = o


@jax.jit
def _run(x_pf, batch_pf, W1, b1, W2, b2, Wc, bc, Wo1, bo1, Wo2, bo2, Wo3, bo3):
    batch = batch_pf.astype(jnp.int32)
    b2d = batch.reshape(N, 1)
    bT = batch.reshape(1, N)
    wcb = jnp.pad(Wc[H:], ((0, 0), (0, _MW - H)))
    wd = Wc[:H] - Wc[H:]

    h, m1, a1 = pl.pallas_call(
        _prep_body,
        out_shape=[jax.ShapeDtypeStruct((N, H), jnp.float32),
                   jax.ShapeDtypeStruct((N, _MW), jnp.float32),
                   jax.ShapeDtypeStruct((N, H), jnp.float32)],
    )(x_pf, W1, b1.reshape(1, -1), W2, b2.reshape(1, -1), wcb, wd,
      bc.reshape(1, -1))

    valid, fetch = _topk_maps(batch)
    topk = _make_topk()
    idx1 = topk(valid, fetch, h, h, b2d, bT)
    f1 = _sc_gather(m1, a1.reshape(-1), idx1.reshape(-1))

    m2, a2 = pl.pallas_call(
        _derive_body,
        out_shape=[jax.ShapeDtypeStruct((N, _MW), jnp.float32),
                   jax.ShapeDtypeStruct((N, H), jnp.float32)],
    )(f1, wcb, wd, bc.reshape(1, -1))
    idx2 = topk(valid, fetch, f1, f1, b2d, bT)
    f2 = _sc_gather(m2, a2.reshape(-1), idx2.reshape(-1))

    o = pl.pallas_call(
        _final_body,
        out_shape=jax.ShapeDtypeStruct((B, 1), jnp.float32),
    )(f2, bT, Wo1, bo1.reshape(1, -1), Wo2, bo2.reshape(1, -1), Wo3,
      bo3.reshape(1, -1))
    return o, jnp.arange(B, dtype=jnp.int32)


def kernel(x_pf, batch_pf, W1, b1, W2, b2, Wc, bc, Wo1, bo1, Wo2, bo2, Wo3,
           bo3):
    return _run(x_pf, batch_pf, W1, b1, W2, b2, Wc, bc, Wo1, bo1, Wo2, bo2,
                Wo3, bo3)
